# baseline (speedup 1.0000x reference)
import functools

import jax
import jax.numpy as jnp
from jax import lax
from jax.experimental import pallas as pl
from jax.experimental.pallas import tpu as pltpu
from jax.experimental.pallas import tpu_sc as plsc

N = 10000
D = 256
H = 128
NH = 2
NC = 2
NS = 16
E2 = 163840
ROWS = E2 // 128
TRASH = N
NP = 10240
NPH = NP // 2
CPT = NPH // NS
MAGIC = 0x5F3759DF

_mesh = plsc.VectorSubcoreMesh(core_axis_name="c", subcore_axis_name="s")


def _prep_body(ei_ref, eff0_ref, eff1_ref, coff_ref):
    r = ei_ref[0]
    c = ei_ref[1]
    eff = jnp.where(r == c, TRASH, r)
    eff0_ref[...] = jnp.where(eff < NPH, eff, NPH)
    eff1_ref[...] = jnp.where(eff >= NPH, eff - NPH, NPH)
    coff_ref[0] = c
    coff_ref[1] = c + NP


def _prep(ei_pad):
    return pl.pallas_call(
        _prep_body,
        out_shape=(
            jax.ShapeDtypeStruct((ROWS, 128), jnp.int32),
            jax.ShapeDtypeStruct((ROWS, 128), jnp.int32),
            jax.ShapeDtypeStruct((NC, ROWS, 128), jnp.int32),
        ),
    )(ei_pad)


@functools.partial(
    pl.kernel,
    out_type=(
        jax.ShapeDtypeStruct((NH * NP, H), jnp.float32),
        jax.ShapeDtypeStruct((NP, 128), jnp.float32),
        jax.ShapeDtypeStruct((NH * NP, H), jnp.float32),
        jax.ShapeDtypeStruct((NH * NP, H), jnp.float32),
    ),
    mesh=_mesh,
    scratch_types=[
        pltpu.VMEM((8, 128), jnp.int32),
        pltpu.VMEM((8, 128), jnp.int32),
        pltpu.VMEM((128, H), jnp.float32),
        pltpu.VMEM((128, H), jnp.float32),
        pltpu.VMEM((2 * CPT // 8, 128), jnp.float32),
        pltpu.SemaphoreType.DMA,
        pltpu.SemaphoreType.DMA,
        pltpu.SemaphoreType.DMA,
        pltpu.SemaphoreType.DMA,
        pltpu.VMEM_SHARED((NPH + 8, H), jnp.float32),
    ],
)
def _mega_kernel(xs_hbm, coff_hbm, eff0_hbm, eff1_hbm,
                 out_hbm, dinv_hbm, g0_hbm, g1_hbm,
                 idx_g, idx_s, bufa, bufb, dv_v,
                 sema, semb, semsa, semsb, acc_sh):
    cid = lax.axis_index("c")
    sid = lax.axis_index("s")
    fbase = cid * NP
    PC = 4
    cbuf = bufa.at[pl.ds(0, 80)]

    def fill(buf, nrows, val):
        def frow(i, _):
            for q in range(H // 16):
                buf[i, pl.ds(q * 16, 16)] = jnp.full((16,), val, jnp.float32)
            return 0

        lax.fori_loop(0, nrows, frow, 0)

    def deg_chunk(k, eff_hbm):
        fill(bufa, 80, 0.0)
        for piece in range(PC):
            pltpu.sync_copy(cbuf,
                            acc_sh.at[pl.ds(sid * CPT + piece * 80, 80)])

        @pl.when(sid == 0)
        def _():
            pltpu.sync_copy(bufa.at[pl.ds(0, 8)], acc_sh.at[pl.ds(NPH, 8)])

        plsc.subcore_barrier()

        def grp(g, _):
            pltpu.sync_copy(eff_hbm.at[pl.ds(sid * 80 + g * 8, 8)], idx_s)

            def fire(r, _):
                pltpu.async_copy(bufb, acc_sh.at[idx_s.at[r]], sema,
                                 add=True)
                return 0

            lax.fori_loop(0, 8, fire, 0)

            def drain(r, _):
                pltpu.make_async_copy(
                    bufb, acc_sh.at[idx_s.at[0]], sema).wait()
                return 0

            lax.fori_loop(0, 8, drain, 0)
            return 0

        lax.fori_loop(0, 10, grp, 0)
        plsc.subcore_barrier()

        for piece in range(PC):
            nbase = sid * CPT + piece * 80
            pltpu.sync_copy(acc_sh.at[pl.ds(nbase, 80)], cbuf)

            def newton(i, _):
                d = bufa[i, pl.ds(0, 16)] + 1.0
                bits = lax.bitcast_convert_type(d, jnp.int32)
                bits = MAGIC - lax.shift_right_logical(bits, 1)
                y = lax.bitcast_convert_type(bits, jnp.float32)
                y = y * (1.5 - 0.5 * d * y * y)
                y = y * (1.5 - 0.5 * d * y * y)
                y = y * (1.5 - 0.5 * d * y * y)
                bufa[i, pl.ds(0, 16)] = y
                soff = (k * CPT + piece * 80) // 8
                dv_v[soff + lax.shift_right_logical(i, 3),
                     pl.ds(jnp.bitwise_and(i, 7) * 16, 16)] = y
                return 0

            lax.fori_loop(0, 80, newton, 0)
            @pl.when(cid == k)
            def _():
                pltpu.sync_copy(cbuf, dinv_hbm.at[pl.ds(k * NPH + nbase, 80)])

    fill(bufb, 128, 1.0)
    deg_chunk(0, eff0_hbm)
    deg_chunk(1, eff1_hbm)

    def scale_rows(koff, power2):
        def srow(i, _):
            v = dv_v[koff // 8 + lax.shift_right_logical(i, 3),
                     pl.ds(jnp.bitwise_and(i, 7) * 16, 16)]
            if power2:
                v = v * v
            for q in range(H // 16):
                bufa[i, pl.ds(q * 16, 16)] = bufa[i, pl.ds(q * 16, 16)] * v
            return 0

        lax.fori_loop(0, 80, srow, 0)

    for k in range(2):
        for piece in range(PC):
            nbase = k * NPH + sid * CPT + piece * 80
            pltpu.sync_copy(xs_hbm.at[pl.ds(fbase + nbase, 80)], cbuf)
            scale_rows(k * CPT + piece * 80, False)
            pltpu.sync_copy(cbuf, g0_hbm.at[pl.ds(fbase + nbase, 80)])

    plsc.subcore_barrier()

    def chunk_pass(src_hbm, dst_hbm, k, eff_hbm, do_scale):
        for piece in range(PC):
            nbase = k * NPH + sid * CPT + piece * 80
            pltpu.sync_copy(src_hbm.at[pl.ds(fbase + nbase, 80)], cbuf)
            pltpu.sync_copy(cbuf,
                            acc_sh.at[pl.ds(sid * CPT + piece * 80, 80)])
        plsc.subcore_barrier()

        def g_start(r, buf, sem):
            pltpu.async_copy(src_hbm.at[idx_g.at[r]], buf, sem)

        def g_wait(buf, sem):
            pltpu.make_async_copy(src_hbm.at[idx_g.at[0]], buf, sem).wait()

        def s_start(r, buf, sem):
            pltpu.async_copy(buf, acc_sh.at[idx_s.at[r]], sem, add=True)

        def s_drain(buf, sem):
            pltpu.make_async_copy(buf, acc_sh.at[idx_s.at[0]], sem).wait()

        def grp(g, _):
            goff = sid * 80 + g * 8
            pltpu.sync_copy(coff_hbm.at[cid].at[pl.ds(goff, 8)], idx_g)
            pltpu.sync_copy(eff_hbm.at[pl.ds(goff, 8)], idx_s)
            g_start(0, bufa, sema)
            g_start(1, bufb, semb)

            def inner(h, _):
                r0 = 2 * h
                g_wait(bufa, sema)
                s_start(r0, bufa, semsa)
                g_wait(bufb, semb)
                s_start(r0 + 1, bufb, semsb)
                s_drain(bufa, semsa)

                @pl.when(h < 3)
                def _():
                    g_start(r0 + 2, bufa, sema)

                s_drain(bufb, semsb)

                @pl.when(h < 3)
                def _():
                    g_start(r0 + 3, bufb, semb)

                return 0

            lax.fori_loop(0, 4, inner, 0)
            return 0

        lax.fori_loop(0, 10, grp, 0)
        plsc.subcore_barrier()

        for piece in range(PC):
            nbase = k * NPH + sid * CPT + piece * 80
            pltpu.sync_copy(acc_sh.at[pl.ds(sid * CPT + piece * 80, 80)],
                            cbuf)
            if do_scale:
                scale_rows(k * CPT + piece * 80, True)
            pltpu.sync_copy(cbuf, dst_hbm.at[pl.ds(fbase + nbase, 80)])

    chunk_pass(g0_hbm, g1_hbm, 0, eff0_hbm, True)
    chunk_pass(g0_hbm, g1_hbm, 1, eff1_hbm, True)
    plsc.subcore_barrier()
    chunk_pass(g1_hbm, out_hbm, 0, eff0_hbm, False)
    chunk_pass(g1_hbm, out_hbm, 1, eff1_hbm, False)


def _final_body(acc_ref, dinv_ref, w_ref, b_ref, out_ref):
    h = jnp.concatenate([acc_ref[q] for q in range(NH)], axis=1)
    h = h * dinv_ref[:, 0:1]
    out_ref[...] = lax.dot_general(
        h, w_ref[...], (((1,), (1,)), ((), ())),
        preferred_element_type=jnp.float32) + b_ref[...]


def _final(acc, dinv, W, b2):
    bn = 1280
    return pl.pallas_call(
        _final_body,
        grid=(NP // bn,),
        in_specs=[
            pl.BlockSpec((NH, bn, H), lambda i: (0, i, 0)),
            pl.BlockSpec((bn, 128), lambda i: (i, 0)),
            pl.BlockSpec((D, D), lambda i: (0, 0)),
            pl.BlockSpec((1, D), lambda i: (0, 0)),
        ],
        out_specs=pl.BlockSpec((bn, D), lambda i: (i, 0)),
        out_shape=jax.ShapeDtypeStruct((NP, D), jnp.float32),
    )(acc, dinv, W, b2)


def kernel(x, edge_index, W, b):
    E = edge_index.shape[1]
    ei_pad = jnp.pad(edge_index, ((0, 0), (0, E2 - E))).reshape(2, ROWS, 128)
    x_pad = jnp.pad(x, ((0, NP - N), (0, 0)))
    xs = jnp.concatenate([x_pad[:, :H], x_pad[:, H:]], axis=0)
    eff0, eff1, coff = _prep(ei_pad)
    acc2, dinv, _, _ = _mega_kernel(xs, coff, eff0, eff1)
    return _final(acc2.reshape(NH, NP, H), dinv, W, b.reshape(1, D))[:N]


# speedup vs baseline: 1.0202x; 1.0202x over previous
import functools

import jax
import jax.numpy as jnp
from jax import lax
from jax.experimental import pallas as pl
from jax.experimental.pallas import tpu as pltpu
from jax.experimental.pallas import tpu_sc as plsc

N = 10000
D = 256
H = 128
NH = 2
NC = 2
NS = 16
E2 = 163840
ROWS = E2 // 128
TRASH = N
NP = 10240
NPH = NP // 2
CPT = NPH // NS
MAGIC = 0x5F3759DF

_mesh = plsc.VectorSubcoreMesh(core_axis_name="c", subcore_axis_name="s")


def _prep_body(ei_ref, eff0_ref, eff1_ref, coff_ref):
    r = ei_ref[0]
    c = ei_ref[1]
    eff = jnp.where(r == c, TRASH, r)
    eff0_ref[...] = jnp.where(eff < NPH, eff, NPH)
    eff1_ref[...] = jnp.where(eff >= NPH, eff - NPH, NPH)
    coff_ref[0] = c
    coff_ref[1] = c + NP


def _prep(ei_pad):
    return pl.pallas_call(
        _prep_body,
        out_shape=(
            jax.ShapeDtypeStruct((ROWS, 128), jnp.int32),
            jax.ShapeDtypeStruct((ROWS, 128), jnp.int32),
            jax.ShapeDtypeStruct((NC, ROWS, 128), jnp.int32),
        ),
    )(ei_pad)


@functools.partial(
    pl.kernel,
    out_type=(
        jax.ShapeDtypeStruct((NH * NP, H), jnp.float32),
        jax.ShapeDtypeStruct((NP, 128), jnp.float32),
        jax.ShapeDtypeStruct((NH * NP, H), jnp.float32),
        jax.ShapeDtypeStruct((NH * NP, H), jnp.float32),
    ),
    mesh=_mesh,
    scratch_types=[
        pltpu.VMEM((16, 128), jnp.int32),
        pltpu.VMEM((16, 128), jnp.int32),
        pltpu.VMEM((128, H), jnp.float32),
        pltpu.VMEM((128, H), jnp.float32),
        pltpu.VMEM((2 * CPT // 8, 128), jnp.float32),
        pltpu.SemaphoreType.DMA,
        pltpu.SemaphoreType.DMA,
        pltpu.SemaphoreType.DMA,
        pltpu.SemaphoreType.DMA,
        pltpu.SemaphoreType.DMA,
        pltpu.SemaphoreType.DMA,
        pltpu.VMEM_SHARED((NPH + 8, H), jnp.float32),
    ],
)
def _mega_kernel(xs_hbm, coff_hbm, eff0_hbm, eff1_hbm,
                 out_hbm, dinv_hbm, g0_hbm, g1_hbm,
                 idx_g, idx_s, bufa, bufb, dv_v,
                 sema, semb, semsa, semsb, semi0, semi1, acc_sh):
    cid = lax.axis_index("c")
    sid = lax.axis_index("s")
    fbase = cid * NP
    PC = 4
    cbuf = bufa.at[pl.ds(0, 80)]

    def fill(buf, nrows, val):
        def frow(i, _):
            for q in range(H // 16):
                buf[i, pl.ds(q * 16, 16)] = jnp.full((16,), val, jnp.float32)
            return 0

        lax.fori_loop(0, nrows, frow, 0)

    def deg_chunk(k, eff_hbm):
        fill(bufa, 80, 0.0)
        for piece in range(PC):
            pltpu.sync_copy(cbuf,
                            acc_sh.at[pl.ds(sid * CPT + piece * 80, 80)])

        @pl.when(sid == 0)
        def _():
            pltpu.sync_copy(bufa.at[pl.ds(0, 8)], acc_sh.at[pl.ds(NPH, 8)])

        plsc.subcore_barrier()

        def load_s(g, half, sem):
            pltpu.async_copy(eff_hbm.at[pl.ds(sid * 80 + g * 8, 8)],
                             idx_s.at[pl.ds(half * 8, 8)], sem)

        def wait_s(half, sem):
            pltpu.make_async_copy(eff_hbm.at[pl.ds(sid * 80, 8)],
                                  idx_s.at[pl.ds(half * 8, 8)], sem).wait()

        def fire8(half):
            def fire(r, _):
                pltpu.async_copy(bufb, acc_sh.at[idx_s.at[half * 8 + r]],
                                 sema, add=True)
                return 0

            lax.fori_loop(0, 8, fire, 0)

            def drain(r, _):
                pltpu.make_async_copy(
                    bufb, acc_sh.at[idx_s.at[0]], sema).wait()
                return 0

            lax.fori_loop(0, 8, drain, 0)

        load_s(0, 0, semi0)

        def pair(p, _):
            g0 = 2 * p
            wait_s(0, semi0)
            load_s(g0 + 1, 1, semi1)
            fire8(0)
            wait_s(1, semi1)

            @pl.when(p < 4)
            def _():
                load_s(g0 + 2, 0, semi0)

            fire8(1)
            return 0

        lax.fori_loop(0, 5, pair, 0)
        plsc.subcore_barrier()

        for piece in range(PC):
            nbase = sid * CPT + piece * 80
            pltpu.sync_copy(acc_sh.at[pl.ds(nbase, 80)], cbuf)

            def newton(i, _):
                d = bufa[i, pl.ds(0, 16)] + 1.0
                bits = lax.bitcast_convert_type(d, jnp.int32)
                bits = MAGIC - lax.shift_right_logical(bits, 1)
                y = lax.bitcast_convert_type(bits, jnp.float32)
                y = y * (1.5 - 0.5 * d * y * y)
                y = y * (1.5 - 0.5 * d * y * y)
                y = y * (1.5 - 0.5 * d * y * y)
                bufa[i, pl.ds(0, 16)] = y
                soff = (k * CPT + piece * 80) // 8
                dv_v[soff + lax.shift_right_logical(i, 3),
                     pl.ds(jnp.bitwise_and(i, 7) * 16, 16)] = y
                return 0

            lax.fori_loop(0, 80, newton, 0)
            @pl.when(cid == k)
            def _():
                pltpu.sync_copy(cbuf, dinv_hbm.at[pl.ds(k * NPH + nbase, 80)])

    fill(bufb, 128, 1.0)
    deg_chunk(0, eff0_hbm)
    deg_chunk(1, eff1_hbm)

    def scale_rows(koff, power2):
        def srow(i, _):
            v = dv_v[koff // 8 + lax.shift_right_logical(i, 3),
                     pl.ds(jnp.bitwise_and(i, 7) * 16, 16)]
            if power2:
                v = v * v
            for q in range(H // 16):
                bufa[i, pl.ds(q * 16, 16)] = bufa[i, pl.ds(q * 16, 16)] * v
            return 0

        lax.fori_loop(0, 80, srow, 0)

    for k in range(2):
        for piece in range(PC):
            nbase = k * NPH + sid * CPT + piece * 80
            pltpu.sync_copy(xs_hbm.at[pl.ds(fbase + nbase, 80)], cbuf)
            scale_rows(k * CPT + piece * 80, False)
            pltpu.sync_copy(cbuf, g0_hbm.at[pl.ds(fbase + nbase, 80)])

    plsc.subcore_barrier()

    def chunk_pass(src_hbm, dst_hbm, k, eff_hbm, do_scale):
        for piece in range(PC):
            nbase = k * NPH + sid * CPT + piece * 80
            pltpu.sync_copy(src_hbm.at[pl.ds(fbase + nbase, 80)], cbuf)
            pltpu.sync_copy(cbuf,
                            acc_sh.at[pl.ds(sid * CPT + piece * 80, 80)])
        plsc.subcore_barrier()

        def g_start(r, buf, sem):
            pltpu.async_copy(src_hbm.at[idx_g.at[r]], buf, sem)

        def g_wait(buf, sem):
            pltpu.make_async_copy(src_hbm.at[idx_g.at[0]], buf, sem).wait()

        def s_start(r, buf, sem):
            pltpu.async_copy(buf, acc_sh.at[idx_s.at[r]], sem, add=True)

        def s_drain(buf, sem):
            pltpu.make_async_copy(buf, acc_sh.at[idx_s.at[0]], sem).wait()

        def load_idx(g, half, sem):
            goff = sid * 80 + g * 8
            off = pl.ds(half * 8, 8)
            pltpu.async_copy(coff_hbm.at[cid].at[pl.ds(goff, 8)],
                             idx_g.at[off], sem)
            pltpu.async_copy(eff_hbm.at[pl.ds(goff, 8)], idx_s.at[off], sem)

        def wait_idx(half, sem):
            off = pl.ds(half * 8, 8)
            pltpu.make_async_copy(coff_hbm.at[cid].at[pl.ds(sid * 80, 8)],
                                  idx_g.at[off], sem).wait()
            pltpu.make_async_copy(eff_hbm.at[pl.ds(sid * 80, 8)],
                                  idx_s.at[off], sem).wait()

        def group8(half):
            base8 = half * 8
            g_start(base8, bufa, sema)
            g_start(base8 + 1, bufb, semb)

            def inner(h, _):
                r0 = base8 + 2 * h
                g_wait(bufa, sema)
                s_start(r0, bufa, semsa)
                g_wait(bufb, semb)
                s_start(r0 + 1, bufb, semsb)
                s_drain(bufa, semsa)

                @pl.when(h < 3)
                def _():
                    g_start(r0 + 2, bufa, sema)

                s_drain(bufb, semsb)

                @pl.when(h < 3)
                def _():
                    g_start(r0 + 3, bufb, semb)

                return 0

            lax.fori_loop(0, 4, inner, 0)

        load_idx(0, 0, semi0)

        def pair(p, _):
            g0 = 2 * p
            wait_idx(0, semi0)
            load_idx(g0 + 1, 1, semi1)
            group8(0)
            wait_idx(1, semi1)

            @pl.when(p < 4)
            def _():
                load_idx(g0 + 2, 0, semi0)

            group8(1)
            return 0

        lax.fori_loop(0, 5, pair, 0)
        plsc.subcore_barrier()

        for piece in range(PC):
            nbase = k * NPH + sid * CPT + piece * 80
            pltpu.sync_copy(acc_sh.at[pl.ds(sid * CPT + piece * 80, 80)],
                            cbuf)
            if do_scale:
                scale_rows(k * CPT + piece * 80, True)
            pltpu.sync_copy(cbuf, dst_hbm.at[pl.ds(fbase + nbase, 80)])

    chunk_pass(g0_hbm, g1_hbm, 0, eff0_hbm, True)
    chunk_pass(g0_hbm, g1_hbm, 1, eff1_hbm, True)
    plsc.subcore_barrier()
    chunk_pass(g1_hbm, out_hbm, 0, eff0_hbm, False)
    chunk_pass(g1_hbm, out_hbm, 1, eff1_hbm, False)


def _final_body(acc_ref, dinv_ref, w_ref, b_ref, out_ref):
    h = jnp.concatenate([acc_ref[q] for q in range(NH)], axis=1)
    h = h * dinv_ref[:, 0:1]
    out_ref[...] = lax.dot_general(
        h, w_ref[...], (((1,), (1,)), ((), ())),
        preferred_element_type=jnp.float32) + b_ref[...]


def _final(acc, dinv, W, b2):
    bn = 1280
    return pl.pallas_call(
        _final_body,
        grid=(NP // bn,),
        in_specs=[
            pl.BlockSpec((NH, bn, H), lambda i: (0, i, 0)),
            pl.BlockSpec((bn, 128), lambda i: (i, 0)),
            pl.BlockSpec((D, D), lambda i: (0, 0)),
            pl.BlockSpec((1, D), lambda i: (0, 0)),
        ],
        out_specs=pl.BlockSpec((bn, D), lambda i: (i, 0)),
        out_shape=jax.ShapeDtypeStruct((NP, D), jnp.float32),
    )(acc, dinv, W, b2)


def kernel(x, edge_index, W, b):
    E = edge_index.shape[1]
    ei_pad = jnp.pad(edge_index, ((0, 0), (0, E2 - E))).reshape(2, ROWS, 128)
    x_pad = jnp.pad(x, ((0, NP - N), (0, 0)))
    xs = jnp.concatenate([x_pad[:, :H], x_pad[:, H:]], axis=0)
    eff0, eff1, coff = _prep(ei_pad)
    acc2, dinv, _, _ = _mega_kernel(xs, coff, eff0, eff1)
    return _final(acc2.reshape(NH, NP, H), dinv, W, b.reshape(1, D))[:N]


# speedup vs baseline: 1.1438x; 1.1212x over previous
import functools

import jax
import jax.numpy as jnp
from jax import lax
from jax.experimental import pallas as pl
from jax.experimental.pallas import tpu as pltpu
from jax.experimental.pallas import tpu_sc as plsc

N = 10000
D = 256
H = 128
NH = 2
NC = 2
NS = 16
E2 = 163840
ROWS = E2 // 128
TRASH = N
NP = 10240
NPH = NP // 2
CPT = NPH // NS
MAGIC = 0x5F3759DF

_mesh = plsc.VectorSubcoreMesh(core_axis_name="c", subcore_axis_name="s")


def _prep_body(ei_ref, eff_ref, eff0_ref, eff1_ref, coff_ref):
    r = ei_ref[0]
    c = ei_ref[1]
    eff = jnp.where(r == c, TRASH, r)
    eff_ref[...] = eff
    eff0_ref[...] = jnp.where(eff < NPH, eff, NPH)
    eff1_ref[...] = jnp.where(eff >= NPH, eff - NPH, NPH)
    coff_ref[0] = c
    coff_ref[1] = c + NP


def _prep(ei_pad):
    return pl.pallas_call(
        _prep_body,
        out_shape=(
            jax.ShapeDtypeStruct((ROWS, 128), jnp.int32),
            jax.ShapeDtypeStruct((ROWS, 128), jnp.int32),
            jax.ShapeDtypeStruct((ROWS, 128), jnp.int32),
            jax.ShapeDtypeStruct((NC, ROWS, 128), jnp.int32),
        ),
    )(ei_pad)


def _deg_body(eff_ref, dinv_ref, acc_ref):
    pid = pl.program_id(0)

    @pl.when(pid == 0)
    def _():
        acc_ref[...] = jnp.zeros((NP // 128, 128), jnp.float32)

    eff = eff_ref[...]
    hi = lax.shift_right_logical(eff, 7)
    lo = jnp.bitwise_and(eff, 127)
    acc = acc_ref[...]
    for r in range(eff.shape[0]):
        oh_hi = (hi[r][:, None] ==
                 lax.broadcasted_iota(jnp.int32, (1, NP // 128), 1)
                 ).astype(jnp.float32)
        oh_lo = (lo[r][:, None] ==
                 lax.broadcasted_iota(jnp.int32, (1, 128), 1)
                 ).astype(jnp.float32)
        acc += lax.dot_general(
            oh_hi, oh_lo, (((0,), (0,)), ((), ())),
            preferred_element_type=jnp.float32)
    acc_ref[...] = acc

    @pl.when(pid == pl.num_programs(0) - 1)
    def _():
        dinv_ref[...] = lax.rsqrt(acc_ref[...] + 1.0)


def _deg_tc(eff):
    eb = 16
    return pl.pallas_call(
        _deg_body,
        grid=(ROWS // eb,),
        in_specs=[pl.BlockSpec((eb, 128), lambda i: (i, 0))],
        out_specs=pl.BlockSpec((NP // 128, 128), lambda i: (0, 0)),
        out_shape=jax.ShapeDtypeStruct((NP // 128, 128), jnp.float32),
        scratch_shapes=[pltpu.VMEM((NP // 128, 128), jnp.float32)],
    )(eff)


@functools.partial(
    pl.kernel,
    out_type=(
        jax.ShapeDtypeStruct((NH * NP, H), jnp.float32),
        jax.ShapeDtypeStruct((NH * NP, H), jnp.float32),
        jax.ShapeDtypeStruct((NH * NP, H), jnp.float32),
    ),
    mesh=_mesh,
    scratch_types=[
        pltpu.VMEM((16, 128), jnp.int32),
        pltpu.VMEM((16, 128), jnp.int32),
        pltpu.VMEM((128, H), jnp.float32),
        pltpu.VMEM((128, H), jnp.float32),
        pltpu.VMEM((2 * CPT // 8, 128), jnp.float32),
        pltpu.SemaphoreType.DMA,
        pltpu.SemaphoreType.DMA,
        pltpu.SemaphoreType.DMA,
        pltpu.SemaphoreType.DMA,
        pltpu.SemaphoreType.DMA,
        pltpu.SemaphoreType.DMA,
        pltpu.VMEM_SHARED((NPH + 8, H), jnp.float32),
    ],
)
def _mega_kernel(xs_hbm, coff_hbm, eff0_hbm, eff1_hbm, dinvp_hbm,
                 out_hbm, g0_hbm, g1_hbm,
                 idx_g, idx_s, bufa, bufb, dv_v,
                 sema, semb, semsa, semsb, semi0, semi1, acc_sh):
    cid = lax.axis_index("c")
    sid = lax.axis_index("s")
    fbase = cid * NP
    PC = 4
    cbuf = bufa.at[pl.ds(0, 80)]

    pltpu.sync_copy(dinvp_hbm, dv_v)

    def scale_rows(koff, power2):
        def sgroup(gi, _):
            nb = koff + gi * 16
            dvc = dv_v[lax.shift_right_logical(nb, 7),
                       pl.ds(jnp.bitwise_and(nb, 127), 16)]
            if power2:
                dvc = dvc * dvc
            for r in range(16):
                v = jnp.full((16,), dvc[r], jnp.float32)
                i = gi * 16 + r
                for q in range(H // 16):
                    bufa[i, pl.ds(q * 16, 16)] = (
                        bufa[i, pl.ds(q * 16, 16)] * v)
            return 0

        lax.fori_loop(0, 5, sgroup, 0)

    for k in range(2):
        for piece in range(PC):
            nbase = k * NPH + sid * CPT + piece * 80
            pltpu.sync_copy(xs_hbm.at[pl.ds(fbase + nbase, 80)], cbuf)
            scale_rows(nbase, False)
            pltpu.sync_copy(cbuf, g0_hbm.at[pl.ds(fbase + nbase, 80)])

    plsc.subcore_barrier()

    def chunk_pass(src_hbm, dst_hbm, k, eff_hbm, do_scale):
        for piece in range(PC):
            nbase = k * NPH + sid * CPT + piece * 80
            pltpu.sync_copy(src_hbm.at[pl.ds(fbase + nbase, 80)], cbuf)
            pltpu.sync_copy(cbuf,
                            acc_sh.at[pl.ds(sid * CPT + piece * 80, 80)])
        plsc.subcore_barrier()

        def g_start(r, buf, sem):
            pltpu.async_copy(src_hbm.at[idx_g.at[r]], buf, sem)

        def g_wait(buf, sem):
            pltpu.make_async_copy(src_hbm.at[idx_g.at[0]], buf, sem).wait()

        def s_start(r, buf, sem):
            pltpu.async_copy(buf, acc_sh.at[idx_s.at[r]], sem, add=True)

        def s_drain(buf, sem):
            pltpu.make_async_copy(buf, acc_sh.at[idx_s.at[0]], sem).wait()

        def load_idx(g, half, sem):
            goff = sid * 80 + g * 8
            off = pl.ds(half * 8, 8)
            pltpu.async_copy(coff_hbm.at[cid].at[pl.ds(goff, 8)],
                             idx_g.at[off], sem)
            pltpu.async_copy(eff_hbm.at[pl.ds(goff, 8)], idx_s.at[off], sem)

        def wait_idx(half, sem):
            off = pl.ds(half * 8, 8)
            pltpu.make_async_copy(coff_hbm.at[cid].at[pl.ds(sid * 80, 8)],
                                  idx_g.at[off], sem).wait()
            pltpu.make_async_copy(eff_hbm.at[pl.ds(sid * 80, 8)],
                                  idx_s.at[off], sem).wait()

        def group8(half):
            base8 = half * 8
            g_start(base8, bufa, sema)
            g_start(base8 + 1, bufb, semb)

            def inner(h, _):
                r0 = base8 + 2 * h
                g_wait(bufa, sema)
                s_start(r0, bufa, semsa)
                g_wait(bufb, semb)
                s_start(r0 + 1, bufb, semsb)
                s_drain(bufa, semsa)

                @pl.when(h < 3)
                def _():
                    g_start(r0 + 2, bufa, sema)

                s_drain(bufb, semsb)

                @pl.when(h < 3)
                def _():
                    g_start(r0 + 3, bufb, semb)

                return 0

            lax.fori_loop(0, 4, inner, 0)

        load_idx(0, 0, semi0)

        def pair(p, _):
            g0 = 2 * p
            wait_idx(0, semi0)
            load_idx(g0 + 1, 1, semi1)
            group8(0)
            wait_idx(1, semi1)

            @pl.when(p < 4)
            def _():
                load_idx(g0 + 2, 0, semi0)

            group8(1)
            return 0

        lax.fori_loop(0, 5, pair, 0)
        plsc.subcore_barrier()

        for piece in range(PC):
            nbase = k * NPH + sid * CPT + piece * 80
            pltpu.sync_copy(acc_sh.at[pl.ds(sid * CPT + piece * 80, 80)],
                            cbuf)
            if do_scale:
                scale_rows(nbase, True)
            pltpu.sync_copy(cbuf, dst_hbm.at[pl.ds(fbase + nbase, 80)])

    chunk_pass(g0_hbm, g1_hbm, 0, eff0_hbm, True)
    chunk_pass(g0_hbm, g1_hbm, 1, eff1_hbm, True)
    plsc.subcore_barrier()
    chunk_pass(g1_hbm, out_hbm, 0, eff0_hbm, False)
    chunk_pass(g1_hbm, out_hbm, 1, eff1_hbm, False)


def _final_body(acc_ref, dinv_ref, w_ref, b_ref, out_ref):
    h = jnp.concatenate([acc_ref[q] for q in range(NH)], axis=1)
    h = h * dinv_ref[...]
    out_ref[...] = lax.dot_general(
        h, w_ref[...], (((1,), (1,)), ((), ())),
        preferred_element_type=jnp.float32) + b_ref[...]


def _final(acc, dinv, W, b2):
    bn = 1280
    return pl.pallas_call(
        _final_body,
        grid=(NP // bn,),
        in_specs=[
            pl.BlockSpec((NH, bn, H), lambda i: (0, i, 0)),
            pl.BlockSpec((bn, 1), lambda i: (i, 0)),
            pl.BlockSpec((D, D), lambda i: (0, 0)),
            pl.BlockSpec((1, D), lambda i: (0, 0)),
        ],
        out_specs=pl.BlockSpec((bn, D), lambda i: (i, 0)),
        out_shape=jax.ShapeDtypeStruct((NP, D), jnp.float32),
    )(acc, dinv, W, b2)


def kernel(x, edge_index, W, b):
    E = edge_index.shape[1]
    ei_pad = jnp.pad(edge_index, ((0, 0), (0, E2 - E))).reshape(2, ROWS, 128)
    x_pad = jnp.pad(x, ((0, NP - N), (0, 0)))
    xs = jnp.concatenate([x_pad[:, :H], x_pad[:, H:]], axis=0)
    eff, eff0, eff1, coff = _prep(ei_pad)
    dinvp = _deg_tc(eff)
    acc2, _, _ = _mega_kernel(xs, coff, eff0, eff1, dinvp)
    return _final(acc2.reshape(NH, NP, H), dinvp.reshape(NP, 1), W,
                  b.reshape(1, D))[:N]
